# baseline (speedup 1.0000x reference)
import jax
import jax.numpy as jnp
from jax.experimental import pallas as pl
from jax.experimental.pallas import tpu as pltpu

_B_BLK = 8
_W_BLK = 2048


def _body(pos_ref, kv_ref, cache_ref, out_ref):
    j = pl.program_id(1)
    local = pos_ref[0] - j * _W_BLK
    hit = (local >= 0) & (local < _W_BLK)

    @pl.when(hit)
    def _():
        ids = jax.lax.broadcasted_iota(jnp.int32, cache_ref.shape, 1)
        out_ref[...] = jnp.where(ids == local, kv_ref[...], cache_ref[...])

    @pl.when(jnp.logical_not(hit))
    def _():
        out_ref[...] = cache_ref[...]


def kernel(kv, start_pos, kv_cache):
    bsz, _, head = kv.shape
    win = kv_cache.shape[1]
    pos = jnp.reshape(jnp.asarray(start_pos, jnp.int32) % win, (1,))
    cache = kv_cache[:bsz]
    out = pl.pallas_call(
        _body,
        grid=(bsz // _B_BLK, win // _W_BLK),
        out_shape=jax.ShapeDtypeStruct(cache.shape, cache.dtype),
        in_specs=[
            pl.BlockSpec(memory_space=pltpu.SMEM),
            pl.BlockSpec((_B_BLK, 1, head), lambda i, j: (i, 0, 0)),
            pl.BlockSpec((_B_BLK, _W_BLK, head), lambda i, j: (i, j, 0)),
        ],
        out_specs=pl.BlockSpec((_B_BLK, _W_BLK, head), lambda i, j: (i, j, 0)),
        compiler_params=pltpu.CompilerParams(vmem_limit_bytes=128 * 1024 * 1024),
    )(pos, kv, cache)
    return out
